# Initial kernel scaffold; baseline (speedup 1.0000x reference)
import jax
import jax.numpy as jnp
from jax.experimental import pallas as pl


def kernel(cw_idxs, qw_idxs, qw_to_phrases, rw_idxs, word_vectors, proj_w, hwy_gate_w, hwy_gate_b, hwy_trans_w, hwy_trans_b):
    raise NotImplementedError("write your pallas kernel here")


# speedup vs baseline: 2.1334x; 2.1334x over previous
import functools

import jax
import jax.numpy as jnp
from jax import lax
from jax.experimental import pallas as pl
from jax.experimental.pallas import tpu as pltpu
from jax.experimental.pallas import tpu_sc as plsc

_VOCAB = 100000
_D = 64
_H = 64
_B = 1024
_LC = 200
_LQ = 20
_P = 10
_N = _B * (_LC + _LQ)

_NC = 2
_NS = 16
_NW = _NC * _NS
_ROWS_PER_WORKER = _N // _NW
_CHUNK = 128
_NCHUNKS = _ROWS_PER_WORKER // _CHUNK


def _fixup_body(qw_ref, ph_ref, rw_ref, out_ref):
    qw = qw_ref[...]
    ph = ph_ref[...]
    repl = jnp.zeros_like(qw)
    for p in range(_P):
        col = rw_ref[:, p : p + 1]
        repl = jnp.where(ph == p, col, repl)
    out_ref[...] = jnp.where(repl > 0, repl, qw)


def _fixup(qw_idxs, qw_to_phrases, rw_idxs):
    return pl.pallas_call(
        _fixup_body,
        out_shape=jax.ShapeDtypeStruct((_B, _LQ), jnp.int32),
    )(qw_idxs, qw_to_phrases, rw_idxs)


def _transform_body(wv_ref, pw_ref, gw_ref, gb_ref, tw_ref, tb_ref, out_ref):
    e = jnp.dot(
        wv_ref[...], pw_ref[...],
        preferred_element_type=jnp.float32,
        precision=lax.Precision.HIGHEST,
    )
    for i in range(2):
        g = jax.nn.sigmoid(
            jnp.dot(e, gw_ref[i], preferred_element_type=jnp.float32,
                    precision=lax.Precision.HIGHEST)
            + gb_ref[i : i + 1, :]
        )
        t = jax.nn.relu(
            jnp.dot(e, tw_ref[i], preferred_element_type=jnp.float32,
                    precision=lax.Precision.HIGHEST)
            + tb_ref[i : i + 1, :]
        )
        e = g * t + (1.0 - g) * e
    out_ref[...] = jnp.concatenate([e, jnp.zeros_like(e)], axis=1)


_TROWS = 2000


def _transform_table(word_vectors, proj_w, gw, gb, tw, tb):
    grid = _VOCAB // _TROWS
    full = lambda *shape: pl.BlockSpec(shape, lambda i: (0,) * len(shape))
    return pl.pallas_call(
        _transform_body,
        grid=(grid,),
        in_specs=[
            pl.BlockSpec((_TROWS, _D), lambda i: (i, 0)),
            full(_D, _H),
            full(2, _H, _H),
            full(2, _H),
            full(2, _H, _H),
            full(2, _H),
        ],
        out_specs=pl.BlockSpec((_TROWS, 2 * _H), lambda i: (i, 0)),
        out_shape=jax.ShapeDtypeStruct((_VOCAB, 2 * _H), jnp.float32),
    )(word_vectors, proj_w, gw, gb, tw, tb)


def _sc_gather(table, idx):
    mesh = plsc.VectorSubcoreMesh(core_axis_name="c", subcore_axis_name="s")

    @functools.partial(
        pl.kernel,
        mesh=mesh,
        out_type=jax.ShapeDtypeStruct((_N, 2 * _H), jnp.float32),
        scratch_types=[
            pltpu.VMEM((_CHUNK,), jnp.int32),
            pltpu.VMEM((_CHUNK, 2 * _H), jnp.float32),
            pltpu.SemaphoreType.DMA,
        ],
    )
    def k(table_hbm, idx_hbm, out_hbm, idx_v, rows_v, sem):
        wid = lax.axis_index("s") * _NC + lax.axis_index("c")
        base = wid * _ROWS_PER_WORKER

        @pl.loop(0, _NCHUNKS)
        def _(ci):
            off = base + ci * _CHUNK
            pltpu.sync_copy(idx_hbm.at[pl.ds(off, _CHUNK)], idx_v)
            pltpu.async_copy(table_hbm.at[idx_v], rows_v, sem).wait()
            pltpu.sync_copy(rows_v, out_hbm.at[pl.ds(off, _CHUNK)])

    return k(table, idx)


def kernel(cw_idxs, qw_idxs, qw_to_phrases, rw_idxs, word_vectors, proj_w,
           hwy_gate_w, hwy_gate_b, hwy_trans_w, hwy_trans_b):
    cw = cw_idxs.astype(jnp.int32)
    qw = qw_idxs.astype(jnp.int32)
    ph = qw_to_phrases.astype(jnp.int32)
    rw = rw_idxs.astype(jnp.int32)

    new_qw = _fixup(qw, ph, rw)
    table2 = _transform_table(word_vectors, proj_w, hwy_gate_w, hwy_gate_b,
                              hwy_trans_w, hwy_trans_b)
    idx = jnp.concatenate([cw, new_qw], axis=1).reshape(-1)
    out = _sc_gather(table2, idx)
    return out[:, :_H].reshape(_B, _LC + _LQ, _H)


# speedup vs baseline: 2.8665x; 1.3437x over previous
import functools

import jax
import jax.numpy as jnp
from jax import lax
from jax.experimental import pallas as pl
from jax.experimental.pallas import tpu as pltpu
from jax.experimental.pallas import tpu_sc as plsc

_VOCAB = 100000
_D = 64
_H = 64
_B = 1024
_LC = 200
_LQ = 20
_P = 10
_N = _B * (_LC + _LQ)

_NC = 2
_NS = 16
_NW = _NC * _NS
_ROWS_PER_WORKER = _N // _NW
_CHUNK = 128
_NCHUNKS = _ROWS_PER_WORKER // _CHUNK


def _fixup_body(qw_ref, ph_ref, rw_ref, out_ref):
    qw = qw_ref[...]
    ph = ph_ref[...]
    repl = jnp.zeros_like(qw)
    for p in range(_P):
        col = rw_ref[:, p : p + 1]
        repl = jnp.where(ph == p, col, repl)
    out_ref[...] = jnp.where(repl > 0, repl, qw)


def _fixup(qw_idxs, qw_to_phrases, rw_idxs):
    return pl.pallas_call(
        _fixup_body,
        out_shape=jax.ShapeDtypeStruct((_B, _LQ), jnp.int32),
    )(qw_idxs, qw_to_phrases, rw_idxs)


def _transform_body(wv_ref, pw_ref, gw_ref, gb_ref, tw_ref, tb_ref, out_ref):
    e = jnp.dot(
        wv_ref[...], pw_ref[...], preferred_element_type=jnp.float32)
    for i in range(2):
        g = jax.nn.sigmoid(
            jnp.dot(e, gw_ref[i], preferred_element_type=jnp.float32)
            + gb_ref[i : i + 1, :]
        )
        t = jax.nn.relu(
            jnp.dot(e, tw_ref[i], preferred_element_type=jnp.float32)
            + tb_ref[i : i + 1, :]
        )
        e = g * t + (1.0 - g) * e
    out_ref[...] = jnp.concatenate([e, jnp.zeros_like(e)], axis=1)


_TROWS = 2000


def _transform_table(word_vectors, proj_w, gw, gb, tw, tb):
    grid = _VOCAB // _TROWS
    full = lambda *shape: pl.BlockSpec(shape, lambda i: (0,) * len(shape))
    return pl.pallas_call(
        _transform_body,
        grid=(grid,),
        in_specs=[
            pl.BlockSpec((_TROWS, _D), lambda i: (i, 0)),
            full(_D, _H),
            full(2, _H, _H),
            full(2, _H),
            full(2, _H, _H),
            full(2, _H),
        ],
        out_specs=pl.BlockSpec((_TROWS, 2 * _H), lambda i: (i, 0)),
        out_shape=jax.ShapeDtypeStruct((_VOCAB, 2 * _H), jnp.float32),
    )(word_vectors, proj_w, gw, gb, tw, tb)


def _sc_gather(table, idx):
    mesh = plsc.VectorSubcoreMesh(core_axis_name="c", subcore_axis_name="s")

    @functools.partial(
        pl.kernel,
        mesh=mesh,
        out_type=jax.ShapeDtypeStruct((_N, 2 * _H), jnp.float32),
        scratch_types=[
            pltpu.VMEM((_CHUNK,), jnp.int32),
            pltpu.VMEM((_CHUNK, 2 * _H), jnp.float32),
            pltpu.SemaphoreType.DMA,
        ],
    )
    def k(table_hbm, idx_hbm, out_hbm, idx_v, rows_v, sem):
        wid = lax.axis_index("s") * _NC + lax.axis_index("c")
        base = wid * _ROWS_PER_WORKER

        @pl.loop(0, _NCHUNKS)
        def _(ci):
            off = base + ci * _CHUNK
            pltpu.sync_copy(idx_hbm.at[pl.ds(off, _CHUNK)], idx_v)
            pltpu.async_copy(table_hbm.at[idx_v], rows_v, sem).wait()
            pltpu.sync_copy(rows_v, out_hbm.at[pl.ds(off, _CHUNK)])

    return k(table, idx)


def kernel(cw_idxs, qw_idxs, qw_to_phrases, rw_idxs, word_vectors, proj_w,
           hwy_gate_w, hwy_gate_b, hwy_trans_w, hwy_trans_b):
    cw = cw_idxs.astype(jnp.int32)
    qw = qw_idxs.astype(jnp.int32)
    ph = qw_to_phrases.astype(jnp.int32)
    rw = rw_idxs.astype(jnp.int32)

    new_qw = _fixup(qw, ph, rw)
    table2 = _transform_table(word_vectors, proj_w, hwy_gate_w, hwy_gate_b,
                              hwy_trans_w, hwy_trans_b)
    idx = jnp.concatenate([cw, new_qw], axis=1).reshape(-1)
    out = _sc_gather(table2, idx)
    return out[:, :_H].reshape(_B, _LC + _LQ, _H)


# speedup vs baseline: 3.3307x; 1.1619x over previous
import functools

import jax
import jax.numpy as jnp
from jax import lax
from jax.experimental import pallas as pl
from jax.experimental.pallas import tpu as pltpu
from jax.experimental.pallas import tpu_sc as plsc

_VOCAB = 100000
_D = 64
_H = 64
_B = 1024
_LC = 200
_LQ = 20
_P = 10
_N = _B * (_LC + _LQ)

_NC = 2
_NS = 16
_NW = _NC * _NS
_ROWS_PER_WORKER = _N // _NW
_CHUNK = 88
_NCHUNKS = _ROWS_PER_WORKER // _CHUNK
_NBUF = 4


def _fixup_body(qw_ref, ph_ref, rw_ref, out_ref):
    qw = qw_ref[...]
    ph = ph_ref[...]
    repl = jnp.zeros_like(qw)
    for p in range(_P):
        col = rw_ref[:, p : p + 1]
        repl = jnp.where(ph == p, col, repl)
    out_ref[...] = jnp.where(repl > 0, repl, qw)


def _fixup(qw_idxs, qw_to_phrases, rw_idxs):
    return pl.pallas_call(
        _fixup_body,
        out_shape=jax.ShapeDtypeStruct((_B, _LQ), jnp.int32),
    )(qw_idxs, qw_to_phrases, rw_idxs)


def _transform_body(wv_ref, pw_ref, gw_ref, gb_ref, tw_ref, tb_ref, out_ref):
    e = jnp.dot(
        wv_ref[...], pw_ref[...], preferred_element_type=jnp.float32)
    for i in range(2):
        g = jax.nn.sigmoid(
            jnp.dot(e, gw_ref[i], preferred_element_type=jnp.float32)
            + gb_ref[i : i + 1, :]
        )
        t = jax.nn.relu(
            jnp.dot(e, tw_ref[i], preferred_element_type=jnp.float32)
            + tb_ref[i : i + 1, :]
        )
        e = g * t + (1.0 - g) * e
    out_ref[...] = jnp.concatenate([e, jnp.zeros_like(e)], axis=1)


_TROWS = 2000


def _transform_table(word_vectors, proj_w, gw, gb, tw, tb):
    grid = _VOCAB // _TROWS
    full = lambda *shape: pl.BlockSpec(shape, lambda i: (0,) * len(shape))
    return pl.pallas_call(
        _transform_body,
        grid=(grid,),
        in_specs=[
            pl.BlockSpec((_TROWS, _D), lambda i: (i, 0)),
            full(_D, _H),
            full(2, _H, _H),
            full(2, _H),
            full(2, _H, _H),
            full(2, _H),
        ],
        out_specs=pl.BlockSpec((_TROWS, 2 * _H), lambda i: (i, 0)),
        out_shape=jax.ShapeDtypeStruct((_VOCAB, 2 * _H), jnp.float32),
    )(word_vectors, proj_w, gw, gb, tw, tb)


def _sc_gather(table, idx):
    mesh = plsc.VectorSubcoreMesh(core_axis_name="c", subcore_axis_name="s")

    @functools.partial(
        pl.kernel,
        mesh=mesh,
        out_type=jax.ShapeDtypeStruct((_N, 2 * _H), jnp.float32),
        scratch_types=(
            [pltpu.VMEM((_CHUNK,), jnp.int32) for _ in range(_NBUF)]
            + [pltpu.VMEM((_CHUNK, 2 * _H), jnp.float32) for _ in range(_NBUF)]
            + [pltpu.SemaphoreType.DMA for _ in range(2 * _NBUF)]
        ),
    )
    def k(table_hbm, idx_hbm, out_hbm, *scratch):
        idx_v = scratch[:_NBUF]
        rows_v = scratch[_NBUF : 2 * _NBUF]
        gsem = scratch[2 * _NBUF : 3 * _NBUF]
        osem = scratch[3 * _NBUF : 4 * _NBUF]
        wid = lax.axis_index("s") * _NC + lax.axis_index("c")
        base = wid * _ROWS_PER_WORKER

        def start_gather(ci, b):
            off = base + ci * _CHUNK
            pltpu.sync_copy(idx_hbm.at[pl.ds(off, _CHUNK)], idx_v[b])
            pltpu.async_copy(table_hbm.at[idx_v[b]], rows_v[b], gsem[b])

        def wait_gather(b):
            pltpu.make_async_copy(table_hbm.at[idx_v[b]], rows_v[b],
                                  gsem[b]).wait()

        def start_out(ci, b):
            off = base + ci * _CHUNK
            pltpu.async_copy(rows_v[b], out_hbm.at[pl.ds(off, _CHUNK)],
                             osem[b])

        def wait_out(ci, b):
            off = base + ci * _CHUNK
            pltpu.make_async_copy(rows_v[b], out_hbm.at[pl.ds(off, _CHUNK)],
                                  osem[b]).wait()

        for b in range(_NBUF):
            start_gather(b, b)

        @pl.loop(0, _NCHUNKS - _NBUF, step=_NBUF)
        def _(k):
            for b in range(_NBUF):
                wait_gather(b)
                start_out(k + b, b)
            for b in range(_NBUF):
                wait_out(k + b, b)
                start_gather(k + b + _NBUF, b)

        for b in range(_NBUF):
            ci = _NCHUNKS - _NBUF + b
            wait_gather(b)
            start_out(ci, b)
        for b in range(_NBUF):
            wait_out(_NCHUNKS - _NBUF + b, b)

    return k(table, idx)


def kernel(cw_idxs, qw_idxs, qw_to_phrases, rw_idxs, word_vectors, proj_w,
           hwy_gate_w, hwy_gate_b, hwy_trans_w, hwy_trans_b):
    cw = cw_idxs.astype(jnp.int32)
    qw = qw_idxs.astype(jnp.int32)
    ph = qw_to_phrases.astype(jnp.int32)
    rw = rw_idxs.astype(jnp.int32)

    new_qw = _fixup(qw, ph, rw)
    table2 = _transform_table(word_vectors, proj_w, hwy_gate_w, hwy_gate_b,
                              hwy_trans_w, hwy_trans_b)
    idx = jnp.concatenate([cw, new_qw], axis=1).reshape(-1)
    out = _sc_gather(table2, idx)
    return out[:, :_H].reshape(_B, _LC + _LQ, _H)
